# Initial kernel scaffold; baseline (speedup 1.0000x reference)
import functools

import jax
import jax.numpy as jnp
from jax import lax
from jax.experimental import pallas as pl
from jax.experimental.pallas import tpu as pltpu
from jax.experimental.pallas import tpu_sc as plsc

F = 26
V = 100000
E = 32
B = 4096
HID = 128

_info = plsc.get_sparse_core_info()
NC = _info.num_cores
NS = _info.num_subcores
NW = NC * NS
BPW = B // NW
RPW = F * BPW


def _sc_gather(flat_tables, categorical_vars):
    mesh = plsc.VectorSubcoreMesh(core_axis_name="c", subcore_axis_name="s")

    @functools.partial(
        pl.kernel,
        mesh=mesh,
        out_type=jax.ShapeDtypeStruct((B * F, E), jnp.float32),
        scratch_types=[
            pltpu.VMEM((F, BPW), jnp.int32),
            pltpu.VMEM((F, BPW), jnp.int32),
            pltpu.VMEM((RPW, E), jnp.float32),
            pltpu.SemaphoreType.DMA,
        ],
    )
    def k(tbl_hbm, idx_hbm, out_hbm, idx_raw, pidx, rows, sem):
        wid = lax.axis_index("s") * NC + lax.axis_index("c")
        b0 = wid * BPW
        pltpu.sync_copy(idx_hbm.at[:, pl.ds(b0, BPW)], idx_raw)

        iota = lax.iota(jnp.int32, 16)
        n_chunk = BPW // 16

        def tr_body(i, carry):
            f = i // n_chunk
            c = i - f * n_chunk
            v = idx_raw[f, pl.ds(c * 16, 16)]
            bl = c * 16 + iota
            tgt = bl * F + f
            plsc.store_scatter(
                pidx,
                [lax.shift_right_logical(tgt, 7), lax.bitwise_and(tgt, 127)],
                v + f * V,
            )
            return carry

        lax.fori_loop(0, F * n_chunk, tr_body, 0)

        def g_body(j, carry):
            pltpu.make_async_copy(
                tbl_hbm.at[pidx.at[j]], rows.at[pl.ds(j * BPW, BPW)], sem
            ).start()
            return carry

        lax.fori_loop(0, F, g_body, 0)

        def w_body(j, carry):
            pltpu.make_async_copy(
                tbl_hbm.at[pidx.at[j]], rows.at[pl.ds(j * BPW, BPW)], sem
            ).wait()
            return carry

        lax.fori_loop(0, F, w_body, 0)

        pltpu.sync_copy(rows, out_hbm.at[pl.ds(wid * RPW, RPW)])

    return k(flat_tables, categorical_vars)


def _mlp_body(x_ref, w1_ref, b1_ref, w2_ref, b2_ref, o_ref):
    h = jnp.dot(x_ref[...], w1_ref[...], preferred_element_type=jnp.float32)
    h = jnp.maximum(h + b1_ref[...], 0.0)
    o = jnp.dot(h, w2_ref[...], preferred_element_type=jnp.float32)
    o_ref[...] = o + b2_ref[...]


def _mlp(cond, W1, b1, W2, b2):
    nblk = 8
    rows = B // nblk
    return pl.pallas_call(
        _mlp_body,
        grid=(nblk,),
        in_specs=[
            pl.BlockSpec((rows, F * E), lambda i: (i, 0)),
            pl.BlockSpec((F * E, HID), lambda i: (0, 0)),
            pl.BlockSpec((1, HID), lambda i: (0, 0)),
            pl.BlockSpec((HID, E), lambda i: (0, 0)),
            pl.BlockSpec((1, E), lambda i: (0, 0)),
        ],
        out_specs=pl.BlockSpec((rows, E), lambda i: (i, 0)),
        out_shape=jax.ShapeDtypeStruct((B, E), jnp.float32),
    )(cond, W1, b1.reshape(1, HID), W2, b2.reshape(1, E))


def kernel(categorical_vars, tables, W1, b1, W2, b2):
    flat_tables = tables.reshape(F * V, E)
    rows = _sc_gather(flat_tables, categorical_vars)
    cond = rows.reshape(B, F * E)
    return _mlp(cond, W1, b1, W2, b2)


# speedup vs baseline: 2.1017x; 2.1017x over previous
import functools

import jax
import jax.numpy as jnp
from jax import lax
from jax.experimental import pallas as pl
from jax.experimental.pallas import tpu as pltpu
from jax.experimental.pallas import tpu_sc as plsc

F = 26
V = 100000
E = 32
B = 4096
HID = 128

_info = plsc.get_sparse_core_info()
NC = _info.num_cores
NS = _info.num_subcores
NW = NC * NS
BPW = B // NW
RPW = F * BPW


def _sc_gather(flat_tables, categorical_vars):
    mesh = plsc.VectorSubcoreMesh(core_axis_name="c", subcore_axis_name="s")

    @functools.partial(
        pl.kernel,
        mesh=mesh,
        out_type=jax.ShapeDtypeStruct((B, F, E), jnp.float32),
        scratch_types=[
            pltpu.VMEM((F, BPW), jnp.int32),
            pltpu.VMEM((F, BPW), jnp.int32),
            pltpu.VMEM((RPW, E), jnp.float32),
            pltpu.SemaphoreType.DMA,
            pltpu.SemaphoreType.DMA,
        ],
        compiler_params=pltpu.CompilerParams(use_tc_tiling_on_sc=False),
    )
    def k(tbl_hbm, idx_hbm, out_hbm, idx_raw, pidx, rows, gsem, wsem):
        wid = lax.axis_index("s") * NC + lax.axis_index("c")
        b0 = wid * BPW
        pltpu.sync_copy(idx_hbm.at[:, pl.ds(b0, BPW)], idx_raw)

        n_chunk = BPW // 16

        def off_body(i, carry):
            f = i // n_chunk
            c = i - f * n_chunk
            sl = pl.ds(c * 16, 16)
            pidx[f, sl] = idx_raw[f, sl] + f * V
            return carry

        lax.fori_loop(0, F * n_chunk, off_body, 0)

        def g_body(j, carry):
            pltpu.make_async_copy(
                tbl_hbm.at[pidx.at[j]], rows.at[pl.ds(j * BPW, BPW)], gsem
            ).start()
            return carry

        lax.fori_loop(0, F, g_body, 0)

        def gw_body(j, carry):
            pltpu.make_async_copy(
                tbl_hbm.at[pidx.at[j]], rows.at[pl.ds(j * BPW, BPW)], gsem
            ).wait()
            return carry

        lax.fori_loop(0, F, gw_body, 0)

        def w_body(j, carry):
            pltpu.make_async_copy(
                rows.at[pl.ds(j * BPW, BPW)], out_hbm.at[pl.ds(b0, BPW), j], wsem
            ).start()
            return carry

        lax.fori_loop(0, F, w_body, 0)

        def ww_body(j, carry):
            pltpu.make_async_copy(
                rows.at[pl.ds(j * BPW, BPW)], out_hbm.at[pl.ds(b0, BPW), j], wsem
            ).wait()
            return carry

        lax.fori_loop(0, F, ww_body, 0)

    return k(flat_tables, categorical_vars)


def _mlp_body(x_ref, w1_ref, b1_ref, w2_ref, b2_ref, o_ref):
    h = jnp.dot(x_ref[...], w1_ref[...], preferred_element_type=jnp.float32)
    h = jnp.maximum(h + b1_ref[...], 0.0)
    o = jnp.dot(h, w2_ref[...], preferred_element_type=jnp.float32)
    o_ref[...] = o + b2_ref[...]


def _mlp(cond, W1, b1, W2, b2):
    nblk = 8
    rows = B // nblk
    return pl.pallas_call(
        _mlp_body,
        grid=(nblk,),
        in_specs=[
            pl.BlockSpec((rows, F * E), lambda i: (i, 0)),
            pl.BlockSpec((F * E, HID), lambda i: (0, 0)),
            pl.BlockSpec((1, HID), lambda i: (0, 0)),
            pl.BlockSpec((HID, E), lambda i: (0, 0)),
            pl.BlockSpec((1, E), lambda i: (0, 0)),
        ],
        out_specs=pl.BlockSpec((rows, E), lambda i: (i, 0)),
        out_shape=jax.ShapeDtypeStruct((B, E), jnp.float32),
    )(cond, W1, b1.reshape(1, HID), W2, b2.reshape(1, E))


def kernel(categorical_vars, tables, W1, b1, W2, b2):
    flat_tables = tables.reshape(F * V, E)
    emb = _sc_gather(flat_tables, categorical_vars)
    cond = emb.reshape(B, F * E)
    return _mlp(cond, W1, b1, W2, b2)


# speedup vs baseline: 2.2109x; 1.0520x over previous
import functools

import jax
import jax.numpy as jnp
from jax import lax
from jax.experimental import pallas as pl
from jax.experimental.pallas import tpu as pltpu
from jax.experimental.pallas import tpu_sc as plsc

F = 26
V = 100000
E = 32
B = 4096
HID = 128

_info = plsc.get_sparse_core_info()
NC = _info.num_cores
NS = _info.num_subcores
NW = NC * NS
BPW = B // NW
RPW = F * BPW


def _sc_gather(tables, categorical_vars):
    mesh = plsc.VectorSubcoreMesh(core_axis_name="c", subcore_axis_name="s")

    @functools.partial(
        pl.kernel,
        mesh=mesh,
        out_type=jax.ShapeDtypeStruct((B, F * E), jnp.float32),
        scratch_types=[
            pltpu.VMEM((F, BPW), jnp.int32),
            pltpu.VMEM((RPW, E), jnp.float32),
            pltpu.SemaphoreType.DMA,
            pltpu.SemaphoreType.DMA,
        ],
        compiler_params=pltpu.CompilerParams(use_tc_tiling_on_sc=False),
    )
    def k(tbl_hbm, idx_hbm, out_hbm, idx_v, rows, gsem, wsem):
        wid = lax.axis_index("s") * NC + lax.axis_index("c")
        b0 = wid * BPW
        pltpu.sync_copy(idx_hbm.at[:, pl.ds(b0, BPW)], idx_v)

        def g_body(j, carry):
            pltpu.make_async_copy(
                tbl_hbm.at[j].at[idx_v.at[j]],
                rows.at[pl.ds(j * BPW, BPW)],
                gsem,
            ).start()
            return carry

        lax.fori_loop(0, F, g_body, 0)

        def gw_body(j, carry):
            pltpu.make_async_copy(
                tbl_hbm.at[j].at[idx_v.at[j]],
                rows.at[pl.ds(j * BPW, BPW)],
                gsem,
            ).wait()
            return carry

        lax.fori_loop(0, F, gw_body, 0)

        def w_body(j, carry):
            pltpu.make_async_copy(
                rows.at[pl.ds(j * BPW, BPW)],
                out_hbm.at[pl.ds(b0, BPW), pl.ds(j * E, E)],
                wsem,
            ).start()
            return carry

        lax.fori_loop(0, F, w_body, 0)

        def ww_body(j, carry):
            pltpu.make_async_copy(
                rows.at[pl.ds(j * BPW, BPW)],
                out_hbm.at[pl.ds(b0, BPW), pl.ds(j * E, E)],
                wsem,
            ).wait()
            return carry

        lax.fori_loop(0, F, ww_body, 0)

    return k(tables, categorical_vars)


def _mlp_body(x_ref, w1_ref, b1_ref, w2_ref, b2_ref, o_ref):
    h = jnp.dot(x_ref[...], w1_ref[...], preferred_element_type=jnp.float32)
    h = jnp.maximum(h + b1_ref[...], 0.0)
    o = jnp.dot(h, w2_ref[...], preferred_element_type=jnp.float32)
    o_ref[...] = o + b2_ref[...]


def _mlp(cond, W1, b1, W2, b2):
    nblk = 8
    rows = B // nblk
    return pl.pallas_call(
        _mlp_body,
        grid=(nblk,),
        in_specs=[
            pl.BlockSpec((rows, F * E), lambda i: (i, 0)),
            pl.BlockSpec((F * E, HID), lambda i: (0, 0)),
            pl.BlockSpec((1, HID), lambda i: (0, 0)),
            pl.BlockSpec((HID, E), lambda i: (0, 0)),
            pl.BlockSpec((1, E), lambda i: (0, 0)),
        ],
        out_specs=pl.BlockSpec((rows, E), lambda i: (i, 0)),
        out_shape=jax.ShapeDtypeStruct((B, E), jnp.float32),
    )(cond, W1, b1.reshape(1, HID), W2, b2.reshape(1, E))


def kernel(categorical_vars, tables, W1, b1, W2, b2):
    cond = _sc_gather(tables, categorical_vars)
    return _mlp(cond, W1, b1, W2, b2)
